# Initial kernel scaffold; baseline (speedup 1.0000x reference)
import jax
import jax.numpy as jnp
from jax.experimental import pallas as pl


def kernel(x, edge_index, edge_attr, W1, b1, gn1_w, gn1_b, gn1_s, W2, b2, gn2_w, gn2_b, gn2_s, W3, b3):
    raise NotImplementedError("write your pallas kernel here")


# speedup vs baseline: 2.6663x; 2.6663x over previous
import dataclasses
import functools

import jax
import jax.numpy as jnp
from jax import lax
from jax.experimental import pallas as pl
from jax.experimental.pallas import tpu as pltpu
from jax.experimental.pallas import tpu_sc as plsc

N = 10000
E = 160000
EPS = 1e-5
FC = 128
NSUB = 16
NPT = 640
NPS = 624
EPT = E // NSUB
EB = 400
NB = EPT // EB
EBH = 80
NBH = EPT // EBH
N2 = 10240

_MESH = plsc.VectorSubcoreMesh(core_axis_name="c", subcore_axis_name="s")

_CP = pltpu.CompilerParams()
if "needs_layout_passes" in pltpu.CompilerParams.__dataclass_fields__:
    _CP = dataclasses.replace(_CP, needs_layout_passes=False)


def _degcnt(col, ea):
    span = N2 // NSUB

    @functools.partial(
        pl.kernel,
        out_type=(jax.ShapeDtypeStruct((N2,), jnp.float32),
                  jax.ShapeDtypeStruct((N2,), jnp.float32)),
        mesh=_MESH,
        compiler_params=_CP,
        scratch_types=[
            pltpu.VMEM((EB,), jnp.int32),
            pltpu.VMEM((EB,), jnp.float32),
            pltpu.VMEM((EB,), jnp.float32),
            pltpu.VMEM((span,), jnp.float32),
            pltpu.VMEM_SHARED((N2,), jnp.float32),
            pltpu.VMEM_SHARED((N2,), jnp.float32),
        ],
    )
    def k(col_ref, ea_ref, deg_ref, cnt_ref, cidx, eab, oneb, zb, accd, accc):
        ci = lax.axis_index("c")
        si = lax.axis_index("s")

        @pl.when(ci == 0)
        def _():
            z16 = jnp.zeros((16,), jnp.float32)
            o16 = jnp.ones((16,), jnp.float32)

            @pl.loop(0, span // 16)
            def _(i):
                zb[pl.ds(i * 16, 16)] = z16

            @pl.loop(0, EB // 16)
            def _(i):
                oneb[pl.ds(i * 16, 16)] = o16

            pltpu.sync_copy(zb, accd.at[pl.ds(si * span, span)])
            pltpu.sync_copy(zb, accc.at[pl.ds(si * span, span)])
            plsc.subcore_barrier()

            @pl.loop(0, NB)
            def _(bi):
                eb = si * EPT + bi * EB
                pltpu.sync_copy(col_ref.at[pl.ds(eb, EB)], cidx)
                pltpu.sync_copy(ea_ref.at[pl.ds(eb, EB)], eab)
                pltpu.sync_copy(eab, accd.at[cidx], add=True)
                pltpu.sync_copy(oneb, accc.at[cidx], add=True)

            plsc.subcore_barrier()
            pltpu.sync_copy(accd.at[pl.ds(si * span, span)],
                            deg_ref.at[pl.ds(si * span, span)])
            pltpu.sync_copy(accc.at[pl.ds(si * span, span)],
                            cnt_ref.at[pl.ds(si * span, span)])

    return k(col, ea)


def _ab(deg, cnt):
    def body(d_ref, c_ref, a_ref, b_ref):
        d = d_ref[...]
        c = c_ref[...]
        safe = jnp.where(d > 0, d, 1.0)
        a = jnp.where(d > 0, lax.rsqrt(safe), 0.0)
        a_ref[...] = a
        b_ref[...] = a / jnp.maximum(c, 1.0)

    a, b = pl.pallas_call(
        body,
        out_shape=(jax.ShapeDtypeStruct((80, 128), jnp.float32),
                   jax.ShapeDtypeStruct((80, 128), jnp.float32)),
    )(deg.reshape(80, 128), cnt.reshape(80, 128))
    return a.reshape(N2), b.reshape(N2)


def _ew2(row, col, ea, av, bv):
    SP = E // 32
    WB = 1024

    @functools.partial(
        pl.kernel,
        out_type=jax.ShapeDtypeStruct((E,), jnp.float32),
        mesh=_MESH,
        compiler_params=_CP,
        scratch_types=[
            pltpu.VMEM((N2,), jnp.float32),
            pltpu.VMEM((N2,), jnp.float32),
            pltpu.VMEM((WB,), jnp.int32),
            pltpu.VMEM((WB,), jnp.int32),
            pltpu.VMEM((WB,), jnp.float32),
            pltpu.VMEM((WB,), jnp.float32),
        ],
    )
    def k(row_ref, col_ref, ea_ref, a_ref, b_ref, ew_ref, a_v, b_v, rv, cv, ev, ob):
        ci = lax.axis_index("c")
        si = lax.axis_index("s")
        w = si * 2 + ci
        s0 = w * SP
        pltpu.sync_copy(a_ref, a_v)
        pltpu.sync_copy(b_ref, b_v)

        @pl.loop(0, (SP + WB - 1) // WB)
        def _(b):
            o = s0 + jnp.minimum(b * WB, SP - WB)
            pltpu.sync_copy(row_ref.at[pl.ds(o, WB)], rv)
            pltpu.sync_copy(col_ref.at[pl.ds(o, WB)], cv)
            pltpu.sync_copy(ea_ref.at[pl.ds(o, WB)], ev)

            @pl.loop(0, WB // 16)
            def _(g):
                go = g * 16
                ar = plsc.load_gather(a_v, [rv[pl.ds(go, 16)]])
                br = plsc.load_gather(b_v, [cv[pl.ds(go, 16)]])
                ob[pl.ds(go, 16)] = ar * br * ev[pl.ds(go, 16)]

            pltpu.sync_copy(ob, ew_ref.at[pl.ds(o, WB)])

    return k(row, col, ea, av, bv)


def _make_hop(ncf):
    @functools.partial(
        pl.kernel,
        out_type=jax.ShapeDtypeStruct((ncf, N, FC), jnp.float32),
        mesh=_MESH,
        compiler_params=_CP,
        scratch_types=[
            pltpu.VMEM((EBH,), jnp.int32),
            pltpu.VMEM((EBH,), jnp.int32),
            pltpu.VMEM((EBH,), jnp.float32),
            pltpu.VMEM((EBH, FC), jnp.float32),
            pltpu.VMEM_SHARED((N, FC), jnp.float32),
        ],
    )
    def k(h_ref, row_ref, col_ref, ew_ref, r_ref, y_ref,
          ridx, cidx, ewb, gbuf, acc):
        ci = lax.axis_index("c")
        si = lax.axis_index("s")
        rs = pl.ds(jnp.minimum(si * NPS, N - NPT), NPT)

        def chunk(c):
            pltpu.sync_copy(r_ref.at[c, rs, :], acc.at[rs])
            plsc.subcore_barrier()

            @pl.loop(0, NBH)
            def _(bi):
                eb = si * EPT + bi * EBH
                pltpu.sync_copy(row_ref.at[pl.ds(eb, EBH)], ridx)
                pltpu.sync_copy(col_ref.at[pl.ds(eb, EBH)], cidx)
                pltpu.sync_copy(ew_ref.at[pl.ds(eb, EBH)], ewb)
                pltpu.sync_copy(h_ref.at[c].at[ridx], gbuf)

                @pl.loop(0, EBH // 16)
                def _(g):
                    o = g * 16
                    wv16 = ewb[pl.ds(o, 16)]
                    for e in range(16):
                        wv = jnp.full((16,), wv16[e], jnp.float32)
                        for j in range(FC // 16):
                            sl = pl.ds(j * 16, 16)
                            gbuf[o + e, sl] = gbuf[o + e, sl] * wv

                pltpu.sync_copy(gbuf, acc.at[cidx], add=True)

            plsc.subcore_barrier()
            pltpu.sync_copy(acc.at[rs], y_ref.at[c, rs, :])

        if ncf == 1:
            @pl.when(ci == 0)
            def _():
                chunk(0)
        else:
            @pl.loop(0, ncf // 2)
            def _(cc):
                chunk(cc * 2 + ci)

    return k


_hop_cache = {}


def _hop(h3, row, col, ew2, r3=None):
    ncf = h3.shape[0]
    if ncf not in _hop_cache:
        _hop_cache[ncf] = _make_hop(ncf)
    if r3 is None:
        r3 = jnp.zeros(h3.shape, jnp.float32)
    return _hop_cache[ncf](h3, row, col, ew2, r3)


def _mm(xs, W, bias, elu_stats, bn=400):
    n = xs[0].shape[0]
    nk = len(xs)
    fo = W.shape[2]
    grid = (n // bn,)
    in_specs = [pl.BlockSpec((bn, x.shape[1]), lambda i: (i, 0)) for x in xs]
    in_specs.append(pl.BlockSpec(W.shape, lambda i: (0, 0, 0)))
    in_specs.append(pl.BlockSpec((1, fo), lambda i: (0, 0)))
    if elu_stats:
        out_shape = (jax.ShapeDtypeStruct((n, fo), jnp.float32),
                     jax.ShapeDtypeStruct((8, fo), jnp.float32))
        out_specs = (pl.BlockSpec((bn, fo), lambda i: (i, 0)),
                     pl.BlockSpec((8, fo), lambda i: (0, 0)))
    else:
        out_shape = jax.ShapeDtypeStruct((n, fo), jnp.float32)
        out_specs = pl.BlockSpec((bn, fo), lambda i: (i, 0))

    def body(*refs):
        xr = refs[:nk]
        wr = refs[nk]
        br = refs[nk + 1]
        yr = refs[nk + 2]
        acc = br[...]
        for k in range(nk):
            acc = acc + jnp.dot(xr[k][...], wr[k],
                                preferred_element_type=jnp.float32,
                                precision=lax.Precision.HIGHEST)
        if elu_stats:
            sr = refs[nk + 3]
            y = jnp.where(acc > 0, acc, jnp.exp(acc) - 1.0)
            yr[...] = y
            s1 = jnp.sum(y, 0, keepdims=True)
            s2 = jnp.sum(y * y, 0, keepdims=True)
            blk = jnp.concatenate([s1, s2, jnp.zeros((6, fo), jnp.float32)], 0)
            i = pl.program_id(0)

            @pl.when(i == 0)
            def _():
                sr[...] = blk

            @pl.when(i > 0)
            def _():
                sr[...] = sr[...] + blk
        else:
            yr[...] = acc

    return pl.pallas_call(
        body, grid=grid, in_specs=in_specs,
        out_shape=out_shape, out_specs=out_specs,
    )(*xs, W, bias.reshape(1, fo))


def _gnorm(h, st, gw, gb, gs, bn=400):
    n, fo = h.shape
    grid = (n // bn,)

    def body(h_ref, s_ref, w_ref, b_ref, g_ref, y_ref):
        s1 = s_ref[0:1, :]
        s2 = s_ref[1:2, :]
        m = s1 / n
        ex2 = s2 / n
        gsv = g_ref[...]
        var = ex2 - (2.0 * gsv - gsv * gsv) * m * m
        inv = lax.rsqrt(var + EPS)
        y_ref[...] = w_ref[...] * (h_ref[...] - gsv * m) * inv + b_ref[...]

    return pl.pallas_call(
        body, grid=grid,
        in_specs=[
            pl.BlockSpec((bn, fo), lambda i: (i, 0)),
            pl.BlockSpec((8, fo), lambda i: (0, 0)),
            pl.BlockSpec((1, fo), lambda i: (0, 0)),
            pl.BlockSpec((1, fo), lambda i: (0, 0)),
            pl.BlockSpec((1, fo), lambda i: (0, 0)),
        ],
        out_shape=jax.ShapeDtypeStruct((n, fo), jnp.float32),
        out_specs=pl.BlockSpec((bn, fo), lambda i: (i, 0)),
    )(h, st, gw.reshape(1, fo), gb.reshape(1, fo), gs.reshape(1, fo))


def _to3(h):
    n, f = h.shape
    return h.reshape(n, f // FC, FC).transpose(1, 0, 2)


def _fr3(h3):
    ncf, n, fc = h3.shape
    return h3.transpose(1, 0, 2).reshape(n, ncf * fc)


def kernel(x, edge_index, edge_attr, W1, b1, gn1_w, gn1_b, gn1_s,
           W2, b2, gn2_w, gn2_b, gn2_s, W3, b3):
    row = edge_index[0]
    col = edge_index[1]
    deg, cnt = _degcnt(col, edge_attr)
    av, bv = _ab(deg, cnt)
    ew2 = _ew2(row, col, edge_attr, av, bv)

    p1 = _hop(_to3(x), row, col, ew2)
    p2 = _hop(p1, row, col, ew2)
    p3 = _hop(p2, row, col, ew2)
    h, st = _mm([x, _fr3(p1), _fr3(p2), _fr3(p3)], W1, b1, True)
    h = _gnorm(h, st, gn1_w, gn1_b, gn1_s)

    p1 = _hop(_to3(h), row, col, ew2)
    p2 = _hop(p1, row, col, ew2)
    p3 = _hop(p2, row, col, ew2)
    h, st = _mm([h, _fr3(p1), _fr3(p2), _fr3(p3)], W2, b2, True)
    h = _gnorm(h, st, gn2_w, gn2_b, gn2_s)

    w3r = jnp.pad(W3, ((0, 0), (0, 0), (0, FC - W3.shape[2])))
    w3r = w3r.transpose(1, 0, 2).reshape(W3.shape[1], 4 * FC)
    b3r = jnp.concatenate([jnp.pad(b3, (0, FC - b3.shape[0])),
                           jnp.zeros(3 * FC, jnp.float32)])
    q = _mm([h], w3r[None], b3r, False)
    t = q[:, 3 * FC:4 * FC][None]
    for k in (2, 1, 0):
        t = _hop(t, row, col, ew2, r3=q[:, k * FC:(k + 1) * FC][None])
    return t[0][:, :W3.shape[2]]


# speedup vs baseline: 5.3294x; 1.9988x over previous
import dataclasses
import functools

import jax
import jax.numpy as jnp
from jax import lax
from jax.experimental import pallas as pl
from jax.experimental.pallas import tpu as pltpu
from jax.experimental.pallas import tpu_sc as plsc

N = 10000
E = 160000
EPS = 1e-5
FC = 128
NSUB = 16
NPT = 640
NPS = 624
EPT = E // NSUB
EB = 400
NB = EPT // EB
EBH = 80
NBH = EPT // EBH
N2 = 10240

_MESH = plsc.VectorSubcoreMesh(core_axis_name="c", subcore_axis_name="s")

_CP = pltpu.CompilerParams()
if "needs_layout_passes" in pltpu.CompilerParams.__dataclass_fields__:
    _CP = dataclasses.replace(_CP, needs_layout_passes=False)


def _degcnt(col, ea):
    span = N2 // NSUB

    @functools.partial(
        pl.kernel,
        out_type=(jax.ShapeDtypeStruct((N2,), jnp.float32),
                  jax.ShapeDtypeStruct((N2,), jnp.float32)),
        mesh=_MESH,
        compiler_params=_CP,
        scratch_types=[
            pltpu.VMEM((EB,), jnp.int32),
            pltpu.VMEM((EB,), jnp.float32),
            pltpu.VMEM((EB,), jnp.float32),
            pltpu.VMEM((span,), jnp.float32),
            pltpu.VMEM_SHARED((N2,), jnp.float32),
            pltpu.VMEM_SHARED((N2,), jnp.float32),
        ],
    )
    def k(col_ref, ea_ref, deg_ref, cnt_ref, cidx, eab, oneb, zb, accd, accc):
        ci = lax.axis_index("c")
        si = lax.axis_index("s")

        @pl.when(ci == 0)
        def _():
            z16 = jnp.zeros((16,), jnp.float32)
            o16 = jnp.ones((16,), jnp.float32)

            @pl.loop(0, span // 16)
            def _(i):
                zb[pl.ds(i * 16, 16)] = z16

            @pl.loop(0, EB // 16)
            def _(i):
                oneb[pl.ds(i * 16, 16)] = o16

            pltpu.sync_copy(zb, accd.at[pl.ds(si * span, span)])
            pltpu.sync_copy(zb, accc.at[pl.ds(si * span, span)])
            plsc.subcore_barrier()

            @pl.loop(0, NB)
            def _(bi):
                eb = si * EPT + bi * EB
                pltpu.sync_copy(col_ref.at[pl.ds(eb, EB)], cidx)
                pltpu.sync_copy(ea_ref.at[pl.ds(eb, EB)], eab)
                pltpu.sync_copy(eab, accd.at[cidx], add=True)
                pltpu.sync_copy(oneb, accc.at[cidx], add=True)

            plsc.subcore_barrier()
            pltpu.sync_copy(accd.at[pl.ds(si * span, span)],
                            deg_ref.at[pl.ds(si * span, span)])
            pltpu.sync_copy(accc.at[pl.ds(si * span, span)],
                            cnt_ref.at[pl.ds(si * span, span)])

    return k(col, ea)


def _ab(deg, cnt):
    def body(d_ref, c_ref, a_ref, b_ref):
        d = d_ref[...]
        c = c_ref[...]
        safe = jnp.where(d > 0, d, 1.0)
        a = jnp.where(d > 0, lax.rsqrt(safe), 0.0)
        a_ref[...] = a
        b_ref[...] = a / jnp.maximum(c, 1.0)

    a, b = pl.pallas_call(
        body,
        out_shape=(jax.ShapeDtypeStruct((80, 128), jnp.float32),
                   jax.ShapeDtypeStruct((80, 128), jnp.float32)),
    )(deg.reshape(80, 128), cnt.reshape(80, 128))
    return a.reshape(N2), b.reshape(N2)


def _ew2(row, col, ea, av, bv):
    SP = E // 32
    WB = 1024

    @functools.partial(
        pl.kernel,
        out_type=jax.ShapeDtypeStruct((E,), jnp.float32),
        mesh=_MESH,
        compiler_params=_CP,
        scratch_types=[
            pltpu.VMEM((N2,), jnp.float32),
            pltpu.VMEM((N2,), jnp.float32),
            pltpu.VMEM((WB,), jnp.int32),
            pltpu.VMEM((WB,), jnp.int32),
            pltpu.VMEM((WB,), jnp.float32),
            pltpu.VMEM((WB,), jnp.float32),
        ],
    )
    def k(row_ref, col_ref, ea_ref, a_ref, b_ref, ew_ref, a_v, b_v, rv, cv, ev, ob):
        ci = lax.axis_index("c")
        si = lax.axis_index("s")
        w = si * 2 + ci
        s0 = w * SP
        pltpu.sync_copy(a_ref, a_v)
        pltpu.sync_copy(b_ref, b_v)

        @pl.loop(0, (SP + WB - 1) // WB)
        def _(b):
            o = s0 + jnp.minimum(b * WB, SP - WB)
            pltpu.sync_copy(row_ref.at[pl.ds(o, WB)], rv)
            pltpu.sync_copy(col_ref.at[pl.ds(o, WB)], cv)
            pltpu.sync_copy(ea_ref.at[pl.ds(o, WB)], ev)

            @pl.loop(0, WB // 16)
            def _(g):
                go = g * 16
                ar = plsc.load_gather(a_v, [rv[pl.ds(go, 16)]])
                br = plsc.load_gather(b_v, [cv[pl.ds(go, 16)]])
                ob[pl.ds(go, 16)] = ar * br * ev[pl.ds(go, 16)]

            pltpu.sync_copy(ob, ew_ref.at[pl.ds(o, WB)])

    return k(row, col, ea, av, bv)


def _make_hop(ncf):
    @functools.partial(
        pl.kernel,
        out_type=jax.ShapeDtypeStruct((ncf, N, FC), jnp.float32),
        mesh=_MESH,
        compiler_params=_CP,
        scratch_types=[
            pltpu.VMEM((2, EBH), jnp.int32),
            pltpu.VMEM((2, EBH), jnp.int32),
            pltpu.VMEM((2, EBH), jnp.float32),
            pltpu.VMEM((2, EBH, FC), jnp.float32),
            pltpu.VMEM_SHARED((N, FC), jnp.float32),
        ] + [pltpu.SemaphoreType.DMA] * 8,
    )
    def k(h_ref, row_ref, col_ref, ew_ref, r_ref, y_ref,
          ridx2, cidx2, ewb2, gbuf2, acc, *sems):
        ci = lax.axis_index("c")
        si = lax.axis_index("s")
        rs = pl.ds(jnp.minimum(si * NPS, N - NPT), NPT)
        srs, scs, ses, sgs = sems[0:2], sems[2:4], sems[4:6], sems[6:8]

        def ebase(bi):
            return si * EPT + bi * EBH

        def idx_copies(bi, p):
            eb = ebase(bi)
            return (
                pltpu.make_async_copy(row_ref.at[pl.ds(eb, EBH)], ridx2.at[p], srs[p]),
                pltpu.make_async_copy(col_ref.at[pl.ds(eb, EBH)], cidx2.at[p], scs[p]),
                pltpu.make_async_copy(ew_ref.at[pl.ds(eb, EBH)], ewb2.at[p], ses[p]),
            )

        def issue_idx(bi, p):
            for cp in idx_copies(bi, p):
                cp.start()

        def gather_copy(c, p):
            return pltpu.make_async_copy(h_ref.at[c].at[ridx2.at[p]],
                                         gbuf2.at[p], sgs[p])

        def scale(p):
            @pl.loop(0, EBH // 16)
            def _(g):
                o = g * 16
                wv16 = ewb2[p, pl.ds(o, 16)]
                for e in range(16):
                    wv = jnp.full((16,), wv16[e], jnp.float32)
                    for j in range(FC // 16):
                        sl = pl.ds(j * 16, 16)
                        gbuf2[p, o + e, sl] = gbuf2[p, o + e, sl] * wv

        def chunk(c):
            pltpu.sync_copy(r_ref.at[c, rs, :], acc.at[rs])
            plsc.subcore_barrier()

            issue_idx(0, 0)
            issue_idx(1, 1)
            idx_copies(0, 0)[0].wait()
            gather_copy(c, 0).start()

            def step(bi, p, last):
                if not last:
                    idx_copies(bi + 1, p ^ 1)[0].wait()
                    gather_copy(c, p ^ 1).start()
                g = gather_copy(c, p)
                g.wait()
                idx_copies(bi, p)[1].wait()
                idx_copies(bi, p)[2].wait()
                scale(p)
                pltpu.sync_copy(gbuf2.at[p], acc.at[cidx2.at[p]], add=True)

            @pl.loop(0, (NBH - 1) // 2)
            def _(t):
                bi0 = t * 2
                step(bi0, 0, False)
                issue_idx(bi0 + 2, 0)
                step(bi0 + 1, 1, False)

                @pl.when(bi0 + 3 < NBH)
                def _():
                    issue_idx(bi0 + 3, 1)

            step(NBH - 1, 0, True)

            plsc.subcore_barrier()
            pltpu.sync_copy(acc.at[rs], y_ref.at[c, rs, :])

        if ncf == 1:
            @pl.when(ci == 0)
            def _():
                chunk(0)
        else:
            @pl.loop(0, ncf // 2)
            def _(cc):
                chunk(cc * 2 + ci)

    return k


_hop_cache = {}


def _hop(h3, row, col, ew2, r3=None):
    ncf = h3.shape[0]
    if ncf not in _hop_cache:
        _hop_cache[ncf] = _make_hop(ncf)
    if r3 is None:
        r3 = jnp.zeros(h3.shape, jnp.float32)
    return _hop_cache[ncf](h3, row, col, ew2, r3)


def _mm(xs, W, bias, elu_stats, bn=400):
    n = xs[0].shape[0]
    nk = len(xs)
    fo = W.shape[2]
    grid = (n // bn,)
    in_specs = [pl.BlockSpec((bn, x.shape[1]), lambda i: (i, 0)) for x in xs]
    in_specs.append(pl.BlockSpec(W.shape, lambda i: (0, 0, 0)))
    in_specs.append(pl.BlockSpec((1, fo), lambda i: (0, 0)))
    if elu_stats:
        out_shape = (jax.ShapeDtypeStruct((n, fo), jnp.float32),
                     jax.ShapeDtypeStruct((8, fo), jnp.float32))
        out_specs = (pl.BlockSpec((bn, fo), lambda i: (i, 0)),
                     pl.BlockSpec((8, fo), lambda i: (0, 0)))
    else:
        out_shape = jax.ShapeDtypeStruct((n, fo), jnp.float32)
        out_specs = pl.BlockSpec((bn, fo), lambda i: (i, 0))

    def body(*refs):
        xr = refs[:nk]
        wr = refs[nk]
        br = refs[nk + 1]
        yr = refs[nk + 2]
        acc = br[...]
        for k in range(nk):
            acc = acc + jnp.dot(xr[k][...], wr[k],
                                preferred_element_type=jnp.float32,
                                precision=lax.Precision.HIGHEST)
        if elu_stats:
            sr = refs[nk + 3]
            y = jnp.where(acc > 0, acc, jnp.exp(acc) - 1.0)
            yr[...] = y
            s1 = jnp.sum(y, 0, keepdims=True)
            s2 = jnp.sum(y * y, 0, keepdims=True)
            blk = jnp.concatenate([s1, s2, jnp.zeros((6, fo), jnp.float32)], 0)
            i = pl.program_id(0)

            @pl.when(i == 0)
            def _():
                sr[...] = blk

            @pl.when(i > 0)
            def _():
                sr[...] = sr[...] + blk
        else:
            yr[...] = acc

    return pl.pallas_call(
        body, grid=grid, in_specs=in_specs,
        out_shape=out_shape, out_specs=out_specs,
    )(*xs, W, bias.reshape(1, fo))


def _gnorm(h, st, gw, gb, gs, bn=400):
    n, fo = h.shape
    grid = (n // bn,)

    def body(h_ref, s_ref, w_ref, b_ref, g_ref, y_ref):
        s1 = s_ref[0:1, :]
        s2 = s_ref[1:2, :]
        m = s1 / n
        ex2 = s2 / n
        gsv = g_ref[...]
        var = ex2 - (2.0 * gsv - gsv * gsv) * m * m
        inv = lax.rsqrt(var + EPS)
        y_ref[...] = w_ref[...] * (h_ref[...] - gsv * m) * inv + b_ref[...]

    return pl.pallas_call(
        body, grid=grid,
        in_specs=[
            pl.BlockSpec((bn, fo), lambda i: (i, 0)),
            pl.BlockSpec((8, fo), lambda i: (0, 0)),
            pl.BlockSpec((1, fo), lambda i: (0, 0)),
            pl.BlockSpec((1, fo), lambda i: (0, 0)),
            pl.BlockSpec((1, fo), lambda i: (0, 0)),
        ],
        out_shape=jax.ShapeDtypeStruct((n, fo), jnp.float32),
        out_specs=pl.BlockSpec((bn, fo), lambda i: (i, 0)),
    )(h, st, gw.reshape(1, fo), gb.reshape(1, fo), gs.reshape(1, fo))


def _to3(h):
    n, f = h.shape
    return h.reshape(n, f // FC, FC).transpose(1, 0, 2)


def _fr3(h3):
    ncf, n, fc = h3.shape
    return h3.transpose(1, 0, 2).reshape(n, ncf * fc)


def kernel(x, edge_index, edge_attr, W1, b1, gn1_w, gn1_b, gn1_s,
           W2, b2, gn2_w, gn2_b, gn2_s, W3, b3):
    row = edge_index[0]
    col = edge_index[1]
    deg, cnt = _degcnt(col, edge_attr)
    av, bv = _ab(deg, cnt)
    ew2 = _ew2(row, col, edge_attr, av, bv)

    p1 = _hop(_to3(x), row, col, ew2)
    p2 = _hop(p1, row, col, ew2)
    p3 = _hop(p2, row, col, ew2)
    h, st = _mm([x, _fr3(p1), _fr3(p2), _fr3(p3)], W1, b1, True)
    h = _gnorm(h, st, gn1_w, gn1_b, gn1_s)

    p1 = _hop(_to3(h), row, col, ew2)
    p2 = _hop(p1, row, col, ew2)
    p3 = _hop(p2, row, col, ew2)
    h, st = _mm([h, _fr3(p1), _fr3(p2), _fr3(p3)], W2, b2, True)
    h = _gnorm(h, st, gn2_w, gn2_b, gn2_s)

    w3r = jnp.pad(W3, ((0, 0), (0, 0), (0, FC - W3.shape[2])))
    w3r = w3r.transpose(1, 0, 2).reshape(W3.shape[1], 4 * FC)
    b3r = jnp.concatenate([jnp.pad(b3, (0, FC - b3.shape[0])),
                           jnp.zeros(3 * FC, jnp.float32)])
    q = _mm([h], w3r[None], b3r, False)
    t = q[:, 3 * FC:4 * FC][None]
    for k in (2, 1, 0):
        t = _hop(t, row, col, ew2, r3=q[:, k * FC:(k + 1) * FC][None])
    return t[0][:, :W3.shape[2]]


# speedup vs baseline: 6.2224x; 1.1676x over previous
import dataclasses
import functools

import jax
import jax.numpy as jnp
from jax import lax
from jax.experimental import pallas as pl
from jax.experimental.pallas import tpu as pltpu
from jax.experimental.pallas import tpu_sc as plsc

N = 10000
E = 160000
EPS = 1e-5
FC = 128
NSUB = 16
NPT = 640
NPS = 624
EPT = E // NSUB
EB = 400
NB = EPT // EB
EBH = 80
NBH = EPT // EBH
N2 = 10240

_MESH = plsc.VectorSubcoreMesh(core_axis_name="c", subcore_axis_name="s")

_CP = pltpu.CompilerParams()
if "needs_layout_passes" in pltpu.CompilerParams.__dataclass_fields__:
    _CP = dataclasses.replace(_CP, needs_layout_passes=False)


def _degcnt(col, ea):
    span = N2 // NSUB

    @functools.partial(
        pl.kernel,
        out_type=(jax.ShapeDtypeStruct((N2,), jnp.float32),
                  jax.ShapeDtypeStruct((N2,), jnp.float32)),
        mesh=_MESH,
        compiler_params=_CP,
        scratch_types=[
            pltpu.VMEM((EB,), jnp.int32),
            pltpu.VMEM((EB,), jnp.float32),
            pltpu.VMEM((EB,), jnp.float32),
            pltpu.VMEM((span,), jnp.float32),
            pltpu.VMEM_SHARED((N2,), jnp.float32),
            pltpu.VMEM_SHARED((N2,), jnp.float32),
        ],
    )
    def k(col_ref, ea_ref, deg_ref, cnt_ref, cidx, eab, oneb, zb, accd, accc):
        ci = lax.axis_index("c")
        si = lax.axis_index("s")

        @pl.when(ci == 0)
        def _():
            z16 = jnp.zeros((16,), jnp.float32)
            o16 = jnp.ones((16,), jnp.float32)

            @pl.loop(0, span // 16)
            def _(i):
                zb[pl.ds(i * 16, 16)] = z16

            @pl.loop(0, EB // 16)
            def _(i):
                oneb[pl.ds(i * 16, 16)] = o16

            pltpu.sync_copy(zb, accd.at[pl.ds(si * span, span)])
            pltpu.sync_copy(zb, accc.at[pl.ds(si * span, span)])
            plsc.subcore_barrier()

            @pl.loop(0, NB)
            def _(bi):
                eb = si * EPT + bi * EB
                pltpu.sync_copy(col_ref.at[pl.ds(eb, EB)], cidx)
                pltpu.sync_copy(ea_ref.at[pl.ds(eb, EB)], eab)
                pltpu.sync_copy(eab, accd.at[cidx], add=True)
                pltpu.sync_copy(oneb, accc.at[cidx], add=True)

            plsc.subcore_barrier()
            pltpu.sync_copy(accd.at[pl.ds(si * span, span)],
                            deg_ref.at[pl.ds(si * span, span)])
            pltpu.sync_copy(accc.at[pl.ds(si * span, span)],
                            cnt_ref.at[pl.ds(si * span, span)])

    return k(col, ea)


def _ab(deg, cnt):
    def body(d_ref, c_ref, a_ref, b_ref):
        d = d_ref[...]
        c = c_ref[...]
        safe = jnp.where(d > 0, d, 1.0)
        a = jnp.where(d > 0, lax.rsqrt(safe), 0.0)
        a_ref[...] = a
        b_ref[...] = a / jnp.maximum(c, 1.0)

    a, b = pl.pallas_call(
        body,
        out_shape=(jax.ShapeDtypeStruct((80, 128), jnp.float32),
                   jax.ShapeDtypeStruct((80, 128), jnp.float32)),
    )(deg.reshape(80, 128), cnt.reshape(80, 128))
    return a.reshape(N2), b.reshape(N2)


def _ew2(row, col, ea, av, bv):
    SP = E // 32
    WB = 1024

    @functools.partial(
        pl.kernel,
        out_type=jax.ShapeDtypeStruct((E,), jnp.float32),
        mesh=_MESH,
        compiler_params=_CP,
        scratch_types=[
            pltpu.VMEM((N2,), jnp.float32),
            pltpu.VMEM((N2,), jnp.float32),
            pltpu.VMEM((WB,), jnp.int32),
            pltpu.VMEM((WB,), jnp.int32),
            pltpu.VMEM((WB,), jnp.float32),
            pltpu.VMEM((WB,), jnp.float32),
        ],
    )
    def k(row_ref, col_ref, ea_ref, a_ref, b_ref, ew_ref, a_v, b_v, rv, cv, ev, ob):
        ci = lax.axis_index("c")
        si = lax.axis_index("s")
        w = si * 2 + ci
        s0 = w * SP
        pltpu.sync_copy(a_ref, a_v)
        pltpu.sync_copy(b_ref, b_v)

        @pl.loop(0, (SP + WB - 1) // WB)
        def _(b):
            o = s0 + jnp.minimum(b * WB, SP - WB)
            pltpu.sync_copy(row_ref.at[pl.ds(o, WB)], rv)
            pltpu.sync_copy(col_ref.at[pl.ds(o, WB)], cv)
            pltpu.sync_copy(ea_ref.at[pl.ds(o, WB)], ev)

            @pl.loop(0, WB // 16)
            def _(g):
                go = g * 16
                ar = plsc.load_gather(a_v, [rv[pl.ds(go, 16)]])
                br = plsc.load_gather(b_v, [cv[pl.ds(go, 16)]])
                ob[pl.ds(go, 16)] = ar * br * ev[pl.ds(go, 16)]

            pltpu.sync_copy(ob, ew_ref.at[pl.ds(o, WB)])

    return k(row, col, ea, av, bv)


def _make_hop(ncf):
    @functools.partial(
        pl.kernel,
        out_type=jax.ShapeDtypeStruct((ncf, N, FC), jnp.float32),
        mesh=_MESH,
        compiler_params=_CP,
        scratch_types=[
            pltpu.VMEM((3, EBH), jnp.int32),
            pltpu.VMEM((3, EBH), jnp.int32),
            pltpu.VMEM((3, EBH), jnp.float32),
            pltpu.VMEM((3, EBH, FC), jnp.float32),
            pltpu.VMEM_SHARED((N, FC), jnp.float32),
        ] + [pltpu.SemaphoreType.DMA] * 15,
    )
    def k(h_ref, row_ref, col_ref, ew_ref, r_ref, y_ref,
          ridx3, cidx3, ewb3, gbuf3, acc, *sems):
        ci = lax.axis_index("c")
        si = lax.axis_index("s")
        rs = pl.ds(jnp.minimum(si * NPS, N - NPT), NPT)
        sr, sc, se, sg, ss = (sems[0:3], sems[3:6], sems[6:9],
                              sems[9:12], sems[12:15])

        def idx_copies(bi, u):
            eb = si * EPT + bi * EBH
            return (
                pltpu.make_async_copy(row_ref.at[pl.ds(eb, EBH)], ridx3.at[u], sr[u]),
                pltpu.make_async_copy(col_ref.at[pl.ds(eb, EBH)], cidx3.at[u], sc[u]),
                pltpu.make_async_copy(ew_ref.at[pl.ds(eb, EBH)], ewb3.at[u], se[u]),
            )

        def gather_copy(c, u):
            return pltpu.make_async_copy(h_ref.at[c].at[ridx3.at[u]],
                                         gbuf3.at[u], sg[u])

        def scatter_copy(u):
            return pltpu.make_async_copy(gbuf3.at[u], acc.at[cidx3.at[u]], ss[u])

        def scale(u):
            @pl.loop(0, EBH // 16)
            def _(g):
                o = g * 16
                wv16 = ewb3[u, pl.ds(o, 16)]
                for e in range(16):
                    wv = jnp.full((16,), wv16[e], jnp.float32)
                    for j in range(FC // 16):
                        sl = pl.ds(j * 16, 16)
                        gbuf3[u, o + e, sl] = gbuf3[u, o + e, sl] * wv

        def chunk(c):
            pltpu.sync_copy(r_ref.at[c, rs, :], acc.at[rs])
            plsc.subcore_barrier()

            for cp in idx_copies(0, 0):
                cp.start()
            for cp in idx_copies(1, 1):
                cp.start()
            idx_copies(0, 0)[0].wait()
            gather_copy(c, 0).start()

            def step(bi, u, first=False, more_g=True, more_i=True):
                if more_g:
                    idx_copies(bi + 1, (u + 1) % 3)[0].wait()
                    gather_copy(c, (u + 1) % 3).start()
                gather_copy(c, u).wait()
                idx_copies(bi, u)[1].wait()
                idx_copies(bi, u)[2].wait()
                scale(u)
                if not first:
                    scatter_copy((u + 2) % 3).wait()
                if more_i:
                    for cp in idx_copies(bi + 2, (u + 2) % 3):
                        cp.start()
                pltpu.async_copy(gbuf3.at[u], acc.at[cidx3.at[u]], ss[u],
                                 add=True)

            step(0, 0, first=True)
            step(1, 1)
            step(2, 2)

            @pl.loop(1, NBH // 3)
            def _(t):
                step(t * 3, 0)
                step(t * 3 + 1, 1)
                step(t * 3 + 2, 2)

            for bi in range(NBH - NBH % 3, NBH):
                step(bi, bi % 3,
                     more_g=(bi + 1 < NBH), more_i=(bi + 2 < NBH))

            scatter_copy((NBH - 1) % 3).wait()
            plsc.subcore_barrier()
            pltpu.sync_copy(acc.at[rs], y_ref.at[c, rs, :])

        if ncf == 1:
            @pl.when(ci == 0)
            def _():
                chunk(0)
        else:
            @pl.loop(0, ncf // 2)
            def _(cc):
                chunk(cc * 2 + ci)

    return k


_hop_cache = {}


def _hop(h3, row, col, ew2, r3=None):
    ncf = h3.shape[0]
    if ncf not in _hop_cache:
        _hop_cache[ncf] = _make_hop(ncf)
    if r3 is None:
        r3 = jnp.zeros(h3.shape, jnp.float32)
    return _hop_cache[ncf](h3, row, col, ew2, r3)


def _mm(xs, W, bias, elu_stats, bn=400):
    n = xs[0].shape[0]
    nk = len(xs)
    fo = W.shape[2]
    grid = (n // bn,)
    in_specs = [pl.BlockSpec((bn, x.shape[1]), lambda i: (i, 0)) for x in xs]
    in_specs.append(pl.BlockSpec(W.shape, lambda i: (0, 0, 0)))
    in_specs.append(pl.BlockSpec((1, fo), lambda i: (0, 0)))
    if elu_stats:
        out_shape = (jax.ShapeDtypeStruct((n, fo), jnp.float32),
                     jax.ShapeDtypeStruct((8, fo), jnp.float32))
        out_specs = (pl.BlockSpec((bn, fo), lambda i: (i, 0)),
                     pl.BlockSpec((8, fo), lambda i: (0, 0)))
    else:
        out_shape = jax.ShapeDtypeStruct((n, fo), jnp.float32)
        out_specs = pl.BlockSpec((bn, fo), lambda i: (i, 0))

    def body(*refs):
        xr = refs[:nk]
        wr = refs[nk]
        br = refs[nk + 1]
        yr = refs[nk + 2]
        acc = br[...]
        for k in range(nk):
            acc = acc + jnp.dot(xr[k][...], wr[k],
                                preferred_element_type=jnp.float32,
                                precision=lax.Precision.HIGHEST)
        if elu_stats:
            sr = refs[nk + 3]
            y = jnp.where(acc > 0, acc, jnp.exp(acc) - 1.0)
            yr[...] = y
            s1 = jnp.sum(y, 0, keepdims=True)
            s2 = jnp.sum(y * y, 0, keepdims=True)
            blk = jnp.concatenate([s1, s2, jnp.zeros((6, fo), jnp.float32)], 0)
            i = pl.program_id(0)

            @pl.when(i == 0)
            def _():
                sr[...] = blk

            @pl.when(i > 0)
            def _():
                sr[...] = sr[...] + blk
        else:
            yr[...] = acc

    return pl.pallas_call(
        body, grid=grid, in_specs=in_specs,
        out_shape=out_shape, out_specs=out_specs,
    )(*xs, W, bias.reshape(1, fo))


def _gnorm(h, st, gw, gb, gs, bn=400):
    n, fo = h.shape
    grid = (n // bn,)

    def body(h_ref, s_ref, w_ref, b_ref, g_ref, y_ref):
        s1 = s_ref[0:1, :]
        s2 = s_ref[1:2, :]
        m = s1 / n
        ex2 = s2 / n
        gsv = g_ref[...]
        var = ex2 - (2.0 * gsv - gsv * gsv) * m * m
        inv = lax.rsqrt(var + EPS)
        y_ref[...] = w_ref[...] * (h_ref[...] - gsv * m) * inv + b_ref[...]

    return pl.pallas_call(
        body, grid=grid,
        in_specs=[
            pl.BlockSpec((bn, fo), lambda i: (i, 0)),
            pl.BlockSpec((8, fo), lambda i: (0, 0)),
            pl.BlockSpec((1, fo), lambda i: (0, 0)),
            pl.BlockSpec((1, fo), lambda i: (0, 0)),
            pl.BlockSpec((1, fo), lambda i: (0, 0)),
        ],
        out_shape=jax.ShapeDtypeStruct((n, fo), jnp.float32),
        out_specs=pl.BlockSpec((bn, fo), lambda i: (i, 0)),
    )(h, st, gw.reshape(1, fo), gb.reshape(1, fo), gs.reshape(1, fo))


def _to3(h):
    n, f = h.shape
    return h.reshape(n, f // FC, FC).transpose(1, 0, 2)


def _fr3(h3):
    ncf, n, fc = h3.shape
    return h3.transpose(1, 0, 2).reshape(n, ncf * fc)


def kernel(x, edge_index, edge_attr, W1, b1, gn1_w, gn1_b, gn1_s,
           W2, b2, gn2_w, gn2_b, gn2_s, W3, b3):
    row = edge_index[0]
    col = edge_index[1]
    deg, cnt = _degcnt(col, edge_attr)
    av, bv = _ab(deg, cnt)
    ew2 = _ew2(row, col, edge_attr, av, bv)

    p1 = _hop(_to3(x), row, col, ew2)
    p2 = _hop(p1, row, col, ew2)
    p3 = _hop(p2, row, col, ew2)
    h, st = _mm([x, _fr3(p1), _fr3(p2), _fr3(p3)], W1, b1, True)
    h = _gnorm(h, st, gn1_w, gn1_b, gn1_s)

    p1 = _hop(_to3(h), row, col, ew2)
    p2 = _hop(p1, row, col, ew2)
    p3 = _hop(p2, row, col, ew2)
    h, st = _mm([h, _fr3(p1), _fr3(p2), _fr3(p3)], W2, b2, True)
    h = _gnorm(h, st, gn2_w, gn2_b, gn2_s)

    w3r = jnp.pad(W3, ((0, 0), (0, 0), (0, FC - W3.shape[2])))
    w3r = w3r.transpose(1, 0, 2).reshape(W3.shape[1], 4 * FC)
    b3r = jnp.concatenate([jnp.pad(b3, (0, FC - b3.shape[0])),
                           jnp.zeros(3 * FC, jnp.float32)])
    q = _mm([h], w3r[None], b3r, False)
    t = q[:, 3 * FC:4 * FC][None]
    for k in (2, 1, 0):
        t = _hop(t, row, col, ew2, r3=q[:, k * FC:(k + 1) * FC][None])
    return t[0][:, :W3.shape[2]]
